# baseline (speedup 1.0000x reference)
import functools

import jax
import jax.numpy as jnp
from jax.experimental import pallas as pl
from jax.experimental.pallas import tpu as pltpu

_N = 100000
_R = 8
_D = 32
_C = 2000


def _dense_body(s_ref, cnt_ref, x_ref, w_ref, root_ref, b_ref, o_ref, *, relu):
    acc = jnp.dot(x_ref[...], root_ref[...], preferred_element_type=jnp.float32)
    for r in range(_R):
        inv = 1.0 / jnp.maximum(cnt_ref[r, 0, 0, :], 1.0)
        acc += jnp.dot(s_ref[r] * inv[:, None], w_ref[r],
                       preferred_element_type=jnp.float32)
    acc += b_ref[...]
    o_ref[...] = jnp.maximum(acc, 0.0) if relu else acc


def _dense_phase(s3, cnt2, x, w, root, b, relu):
    grid = (_N // _C,)
    return pl.pallas_call(
        functools.partial(_dense_body, relu=relu),
        grid=grid,
        in_specs=[
            pl.BlockSpec((_R, _C, _D), lambda i: (0, i, 0)),
            pl.BlockSpec((_R, 1, 1, _C), lambda i: (0, i, 0, 0)),
            pl.BlockSpec((_C, _D), lambda i: (i, 0)),
            pl.BlockSpec((_R, _D, _D), lambda i: (0, 0, 0)),
            pl.BlockSpec((_D, _D), lambda i: (0, 0)),
            pl.BlockSpec((1, _D), lambda i: (0, 0)),
        ],
        out_specs=pl.BlockSpec((_C, _D), lambda i: (i, 0)),
        out_shape=jax.ShapeDtypeStruct((_N, _D), jnp.float32),
    )(s3, cnt2.reshape(_R, _N // _C, 1, _C), x, w, root, b.reshape(1, _D))


def kernel(x, edge_index, edge_type, W1, root1, b1, W2, root2, b2):
    src = edge_index[0]
    dst = edge_index[1]
    seg = edge_type * _N + dst

    cnt = jax.ops.segment_sum(jnp.ones((src.shape[0],), jnp.float32), seg,
                              num_segments=_N * _R)
    cnt2 = cnt.reshape(_R, _N)

    h = x
    for (w, root, b, relu) in ((W1, root1, b1, True), (W2, root2, b2, False)):
        s = jax.ops.segment_sum(h[src], seg, num_segments=_N * _R)
        h = _dense_phase(s.reshape(_R, _N, _D), cnt2, h, w, root, b, relu)
    return h
